# baseline (speedup 1.0000x reference)
import functools

import jax
import jax.numpy as jnp
from jax.experimental import pallas as pl

NUM_EMBEDDINGS = 1024
EMBEDDING_DIM = 64
SPARSITY = 5
PATCH = 2
COMMIT = 0.25
EPS = 1e-10
ALPHA = 0.3
ATOM_DIM = EMBEDDING_DIM * PATCH * PATCH


def _gram_kernel(x_ref, dn_ref, g_ref, hbar_ref):
    X = x_ref[...]
    Dn = dn_ref[...]
    g_ref[...] = jax.lax.dot_general(
        Dn, Dn, (((0,), (0,)), ((), ())), preferred_element_type=jnp.float32)
    hbar_ref[...] = jax.lax.dot_general(
        X, Dn, (((1,), (0,)), ((), ())), preferred_element_type=jnp.float32)


def _recon_kernel(dn_ref, oh_ref, coef_ref, x_ref, recon_ref, err_ref):
    i = pl.program_id(0)
    Dn = dn_ref[...]
    X = x_ref[...]
    recon = None
    for k in range(SPARSITY):
        oh = oh_ref[k]
        d_k = jax.lax.dot_general(oh, Dn, (((1,), (1,)), ((), ())),
                                  preferred_element_type=jnp.float32)
        c_k = coef_ref[:, k:k + 1]
        term = c_k * d_k
        recon = term if recon is None else recon + term
    recon_ref[...] = recon
    diff = recon - X
    blk = jnp.sum(diff * diff, keepdims=True)

    @pl.when(i == 0)
    def _init():
        err_ref[...] = jnp.zeros_like(err_ref)

    err_ref[...] += blk


@jax.jit
def _dict_forward(z, dictionary, usage_ema):
    Bz, C, H, W = z.shape
    P = PATCH
    Hp, Wp = H // P, W // P
    patches = z.reshape(Bz, C, Hp, P, Wp, P).transpose(0, 2, 4, 1, 3, 5)
    patches = patches.reshape(Bz * Hp * Wp, C * P * P)
    X = patches
    T, M = X.shape
    N = dictionary.shape[1]

    norms = jnp.maximum(jnp.linalg.norm(dictionary, axis=0, keepdims=True), EPS)
    Dn = dictionary / norms

    G, h_bar = pl.pallas_call(
        _gram_kernel,
        out_shape=[
            jax.ShapeDtypeStruct((N, N), jnp.float32),
            jax.ShapeDtypeStruct((T, N), jnp.float32),
        ],
    )(X, Dn)

    usage = usage_ema / jnp.maximum(usage_ema.sum(), EPS)
    uniform = 1.0 / max(1.0, float(N))
    boost = jnp.minimum((uniform / jnp.maximum(usage, EPS)) ** ALPHA, 8.0)

    h = h_bar
    B = T
    L = jnp.ones((B, 1, 1), jnp.float32)
    mask = jnp.ones((B, N), dtype=bool)
    bidx = jnp.arange(B)
    I = None
    gamma_stack = None
    for k in range(1, SPARSITY + 1):
        scores = jnp.abs(h) * mask.astype(jnp.float32) * boost[None, :]
        idx = jnp.argmax(scores, axis=1)
        mask = mask.at[bidx, idx].set(False)
        if k > 1:
            G_col = G[I, idx[:, None]][..., None]
            w = jnp.linalg.solve(L, G_col)
            wT = jnp.swapaxes(w, 1, 2)
            w_corner = jnp.sqrt(jnp.maximum(
                1.0 - jnp.sum(wT ** 2, axis=2, keepdims=True), 1e-12))
            zeros = jnp.zeros((B, k - 1, 1), jnp.float32)
            L = jnp.concatenate([jnp.concatenate([L, zeros], axis=2),
                                 jnp.concatenate([wT, w_corner], axis=2)], axis=1)
            I = jnp.concatenate([I, idx[:, None]], axis=1)
        else:
            I = idx[:, None]
        h_stack = jnp.take_along_axis(h_bar, I, axis=1)[..., None]
        y = jnp.linalg.solve(L, h_stack)
        gamma_stack = jnp.linalg.solve(jnp.swapaxes(L, 1, 2), y)
        if k < SPARSITY:
            beta = jnp.einsum('bk,bkn->bn', gamma_stack[..., 0], G[I])
            h = h_bar - beta

    coeffs = gamma_stack[..., 0]
    onehots = jax.nn.one_hot(I.T, N, dtype=jnp.float32)

    block_b = 512
    recon, err = pl.pallas_call(
        _recon_kernel,
        grid=(T // block_b,),
        in_specs=[
            pl.BlockSpec((M, N), lambda i: (0, 0)),
            pl.BlockSpec((SPARSITY, block_b, N), lambda i: (0, i, 0)),
            pl.BlockSpec((block_b, SPARSITY), lambda i: (i, 0)),
            pl.BlockSpec((block_b, M), lambda i: (i, 0)),
        ],
        out_specs=[
            pl.BlockSpec((block_b, M), lambda i: (i, 0)),
            pl.BlockSpec((1, 1), lambda i: (0, 0)),
        ],
        out_shape=[
            jax.ShapeDtypeStruct((T, M), jnp.float32),
            jax.ShapeDtypeStruct((1, 1), jnp.float32),
        ],
    )(Dn, onehots, coeffs, X)

    loss = (1.0 + COMMIT) * err[0, 0] / (T * M)
    zq = recon.reshape(Bz, Hp, Wp, C, P, P).transpose(0, 3, 1, 4, 2, 5)
    zq = zq.reshape(Bz, C, H, W)
    return zq, loss


def kernel(z, dictionary, usage_ema):
    return _dict_forward(z, dictionary, usage_ema)


# speedup vs baseline: 1.0214x; 1.0214x over previous
import functools

import jax
import jax.numpy as jnp
from jax.experimental import pallas as pl

NUM_EMBEDDINGS = 1024
EMBEDDING_DIM = 64
SPARSITY = 5
PATCH = 2
COMMIT = 0.25
EPS = 1e-10
ALPHA = 0.3
ATOM_DIM = EMBEDDING_DIM * PATCH * PATCH


def _gram_kernel(x_ref, dn_ref, g_ref, hbar_ref):
    X = x_ref[...]
    Dn = dn_ref[...]
    g_ref[...] = jax.lax.dot_general(
        Dn, Dn, (((0,), (0,)), ((), ())), preferred_element_type=jnp.float32)
    hbar_ref[...] = jax.lax.dot_general(
        X, Dn, (((1,), (0,)), ((), ())), preferred_element_type=jnp.float32)


def _bf16_rne(x):
    u = jax.lax.bitcast_convert_type(x, jnp.int32)
    lsb = jax.lax.shift_right_logical(u, 16) & 1
    u = u + 0x7FFF + lsb
    u = u & jnp.int32(-65536)
    return jax.lax.bitcast_convert_type(u, jnp.float32)


def _step_kernel(hbar_ref, g_ref, boost_ref, i_ref, gam_ref, *refs, k, n):
    row_refs = refs[:k - 1]
    if k < SPARSITY:
        idx_ref, hst_ref, rownew_ref = refs[k - 1:]
    else:
        idx_ref, hst_ref = refs[k - 1:]
    h_bar = hbar_ref[...]
    boost = boost_ref[...]
    Bb = h_bar.shape[0]
    iota_n = jax.lax.broadcasted_iota(jnp.int32, (Bb, n), 1)
    if k > 1:
        I = i_ref[...]
        gam = _bf16_rne(gam_ref[...])
        notsel = jnp.ones((Bb, n), jnp.float32)
        s = None
        c = None
        for j in range(k - 1):
            oh = (iota_n == I[:, j:j + 1]).astype(jnp.float32)
            notsel = notsel * (1.0 - oh)
            p = gam[:, j:j + 1] * row_refs[j][...]
            if s is None:
                s = p
                c = jnp.zeros_like(p)
            else:
                t = s + p
                big = jnp.abs(s) >= jnp.abs(p)
                c = c + jnp.where(big, (s - t) + p, (p - t) + s)
                s = t
        beta = s + c
        h = h_bar - beta
    else:
        notsel = jnp.ones((Bb, n), jnp.float32)
        h = h_bar
    scores = jnp.abs(h) * notsel * boost
    m = jnp.max(scores, axis=1, keepdims=True)
    idx = jnp.min(jnp.where(scores == m, iota_n, n), axis=1, keepdims=True)
    onehot = (iota_n == idx).astype(jnp.float32)
    idx_ref[...] = idx
    hst_ref[...] = jnp.sum(h_bar * onehot, axis=1, keepdims=True)
    if k < SPARSITY:
        G = g_ref[...]
        rownew_ref[...] = jax.lax.dot_general(
            onehot, G, (((1,), (0,)), ((), ())),
            preferred_element_type=jnp.float32)


def _recon_kernel(dn_ref, i_ref, coef_ref, x_ref, recon_ref, err_ref, *, n):
    gi = pl.program_id(0)
    Dn = dn_ref[...]
    X = x_ref[...]
    I = i_ref[...]
    Bb = X.shape[0]
    iota_n = jax.lax.broadcasted_iota(jnp.int32, (Bb, n), 1)
    recon = None
    for k in range(SPARSITY):
        oh = (iota_n == I[:, k:k + 1]).astype(jnp.float32)
        d_k = jax.lax.dot_general(oh, Dn, (((1,), (1,)), ((), ())),
                                  preferred_element_type=jnp.float32)
        term = coef_ref[:, k:k + 1] * d_k
        recon = term if recon is None else recon + term
    recon_ref[...] = recon
    diff = recon - X
    blk = jnp.sum(diff * diff, keepdims=True)

    @pl.when(gi == 0)
    def _init():
        err_ref[...] = jnp.zeros_like(err_ref)

    err_ref[...] += blk


@jax.jit
def _dict_forward(z, dictionary, usage_ema):
    Bz, C, H, W = z.shape
    P = PATCH
    Hp, Wp = H // P, W // P
    patches = z.reshape(Bz, C, Hp, P, Wp, P).transpose(0, 2, 4, 1, 3, 5)
    patches = patches.reshape(Bz * Hp * Wp, C * P * P)
    X = patches
    T, M = X.shape
    N = dictionary.shape[1]
    B = T

    norms = jnp.maximum(jnp.linalg.norm(dictionary, axis=0, keepdims=True), EPS)
    Dn = dictionary / norms

    G, h_bar = pl.pallas_call(
        _gram_kernel,
        out_shape=[
            jax.ShapeDtypeStruct((N, N), jnp.float32),
            jax.ShapeDtypeStruct((T, N), jnp.float32),
        ],
    )(X, Dn)

    usage = usage_ema / jnp.maximum(usage_ema.sum(), EPS)
    uniform = 1.0 / max(1.0, float(N))
    boost = jnp.minimum((uniform / jnp.maximum(usage, EPS)) ** ALPHA, 8.0)
    boost2d = boost.reshape(1, N)

    block_b = 512
    grid = (B // block_b,)

    L = jnp.ones((B, 1, 1), jnp.float32)
    I = None
    gamma_stack = None
    hst_list = []
    rows = []
    for k in range(1, SPARSITY + 1):
        kc = max(k - 1, 1)
        I_in = I if k > 1 else jnp.zeros((B, 1), jnp.int32)
        gam_in = gamma_stack[..., 0] if k > 1 else jnp.zeros((B, 1), jnp.float32)
        out_specs = [
            pl.BlockSpec((block_b, 1), lambda i: (i, 0)),
            pl.BlockSpec((block_b, 1), lambda i: (i, 0)),
        ]
        out_shape = [
            jax.ShapeDtypeStruct((B, 1), jnp.int32),
            jax.ShapeDtypeStruct((B, 1), jnp.float32),
        ]
        if k < SPARSITY:
            out_specs.append(pl.BlockSpec((block_b, N), lambda i: (i, 0)))
            out_shape.append(jax.ShapeDtypeStruct((B, N), jnp.float32))
        outs = pl.pallas_call(
            functools.partial(_step_kernel, k=k, n=N),
            grid=grid,
            in_specs=[
                pl.BlockSpec((block_b, N), lambda i: (i, 0)),
                pl.BlockSpec((N, N), lambda i: (0, 0)),
                pl.BlockSpec((1, N), lambda i: (0, 0)),
                pl.BlockSpec((block_b, kc), lambda i: (i, 0)),
                pl.BlockSpec((block_b, kc), lambda i: (i, 0)),
            ] + [pl.BlockSpec((block_b, N), lambda i: (i, 0))] * (k - 1),
            out_specs=out_specs,
            out_shape=out_shape,
        )(h_bar, G, boost2d, I_in, gam_in, *rows)
        if k < SPARSITY:
            idx2d, hst, row_new = outs
            rows.append(row_new)
        else:
            idx2d, hst = outs
        idx = idx2d[:, 0]
        hst_list.append(hst)
        if k > 1:
            G_col = G[I, idx[:, None]][..., None]
            w = jnp.linalg.solve(L, G_col)
            wT = jnp.swapaxes(w, 1, 2)
            w_corner = jnp.sqrt(jnp.maximum(
                1.0 - jnp.sum(wT ** 2, axis=2, keepdims=True), 1e-12))
            zeros = jnp.zeros((B, k - 1, 1), jnp.float32)
            L = jnp.concatenate([jnp.concatenate([L, zeros], axis=2),
                                 jnp.concatenate([wT, w_corner], axis=2)], axis=1)
            I = jnp.concatenate([I, idx[:, None]], axis=1)
        else:
            I = idx[:, None]
        h_stack = jnp.concatenate(hst_list, axis=1)[..., None]
        y = jnp.linalg.solve(L, h_stack)
        gamma_stack = jnp.linalg.solve(jnp.swapaxes(L, 1, 2), y)

    coeffs = gamma_stack[..., 0]

    recon, err = pl.pallas_call(
        functools.partial(_recon_kernel, n=N),
        grid=grid,
        in_specs=[
            pl.BlockSpec((M, N), lambda i: (0, 0)),
            pl.BlockSpec((block_b, SPARSITY), lambda i: (i, 0)),
            pl.BlockSpec((block_b, SPARSITY), lambda i: (i, 0)),
            pl.BlockSpec((block_b, M), lambda i: (i, 0)),
        ],
        out_specs=[
            pl.BlockSpec((block_b, M), lambda i: (i, 0)),
            pl.BlockSpec((1, 1), lambda i: (0, 0)),
        ],
        out_shape=[
            jax.ShapeDtypeStruct((T, M), jnp.float32),
            jax.ShapeDtypeStruct((1, 1), jnp.float32),
        ],
    )(Dn, I, coeffs, X)

    loss = (1.0 + COMMIT) * err[0, 0] / (T * M)
    zq = recon.reshape(Bz, Hp, Wp, C, P, P).transpose(0, 3, 1, 4, 2, 5)
    zq = zq.reshape(Bz, C, H, W)
    return zq, loss


def kernel(z, dictionary, usage_ema):
    return _dict_forward(z, dictionary, usage_ema)
